# baseline (speedup 1.0000x reference)
import functools

import jax
import jax.numpy as jnp
from jax import lax
from jax.experimental import pallas as pl
from jax.experimental.pallas import tpu as pltpu
from jax.experimental.pallas import tpu_sc as plsc

B = 16384
NF = 26
FIELD_SIZE = 100000
NC = 2
NS = 16
NW = NC * NS
ROWS_W = B // NW
CH = 128
CPF = ROWS_W // CH
LANES = 16
DEPTH = 8

FIELD_SPLITS = ((0, 8), (8, 16), (16, 24), (24, 26))


def _emb_partial_sc(xt, tab, f0, f1, row0):
    nf = f1 - f0
    flat = nf * ROWS_W
    nch = flat // CH
    mesh = plsc.VectorSubcoreMesh(
        core_axis_name="c", subcore_axis_name="s", num_cores=NC, num_subcores=NS
    )

    @functools.partial(
        pl.kernel,
        out_type=jax.ShapeDtypeStruct((B,), jnp.float32),
        mesh=mesh,
        compiler_params=pltpu.CompilerParams(use_tc_tiling_on_sc=False),
        scratch_types=[
            pltpu.VMEM((nf, ROWS_W), jnp.int32),
            pltpu.VMEM((flat,), jnp.float32),
            pltpu.VMEM((ROWS_W,), jnp.float32),
            pltpu.SemaphoreType.DMA,
        ],
        name=f"emb_gather_f{f0}_{f1}",
    )
    def k(xt_hbm, tab_hbm, out_hbm, idx_v, rows_v, acc_v, sem):
        wid = lax.axis_index("s") * NC + lax.axis_index("c")
        base = wid * ROWS_W
        tab_flat = tab_hbm.at[0]
        pltpu.sync_copy(xt_hbm.at[pl.ds(f0, nf), pl.ds(base, ROWS_W)], idx_v)

        def add_offsets(f, carry):
            off = (f0 + f) * FIELD_SIZE - row0
            for c in range(ROWS_W // LANES):
                sl = pl.ds(c * LANES, LANES)
                idx_v[f, sl] = idx_v[f, sl] + off
            return carry

        lax.fori_loop(0, nf, add_offsets, 0)

        def src(j):
            return tab_flat.at[idx_v.at[j // CPF, pl.ds((j % CPF) * CH, CH)]]

        def fire(j):
            pltpu.async_copy(src(j), rows_v.at[pl.ds(j * CH, CH)], sem)

        def drain(j):
            pltpu.make_async_copy(src(j), rows_v.at[pl.ds(j * CH, CH)], sem).wait()

        for j in range(DEPTH):
            fire(j)

        def steady(j, carry):
            fire(j + DEPTH)
            drain(j)
            return carry

        lax.fori_loop(0, nch - DEPTH, steady, 0)

        def tail(j, carry):
            drain(j)
            return carry

        lax.fori_loop(nch - DEPTH, nch, tail, 0)

        def row_sum(g, carry):
            acc = rows_v[pl.ds(g * LANES, LANES)]
            for f in range(1, nf):
                acc = acc + rows_v[pl.ds(f * ROWS_W + g * LANES, LANES)]
            acc_v[pl.ds(g * LANES, LANES)] = acc
            return carry

        lax.fori_loop(0, ROWS_W // LANES, row_sum, 0)
        pltpu.sync_copy(acc_v, out_hbm.at[pl.ds(base, ROWS_W)])

    return k(xt, tab)


def _matvec_body(xc_ref, w_ref, b_ref, o_ref):
    o_ref[...] = jnp.sum(xc_ref[...] * w_ref[...], axis=1) + b_ref[0, 0]


def _matvec_tc(x_cont, w2, bias2):
    blk = 2048
    return pl.pallas_call(
        _matvec_body,
        grid=(B // blk,),
        in_specs=[
            pl.BlockSpec((blk, 128), lambda i: (i, 0)),
            pl.BlockSpec((1, 128), lambda i: (0, 0)),
            pl.BlockSpec((1, 1), lambda i: (0, 0)),
        ],
        out_specs=pl.BlockSpec((blk,), lambda i: (i,)),
        out_shape=jax.ShapeDtypeStruct((B,), jnp.float32),
        name="cont_matvec",
    )(x_cont, w2, bias2)


def _combine_body(c_ref, e0_ref, e1_ref, e2_ref, e3_ref, o_ref):
    o_ref[...] = c_ref[...] + e0_ref[...] + e1_ref[...] + e2_ref[...] + e3_ref[...]


def _combine_tc(cont, embs):
    blk = 4096
    vec = pl.BlockSpec((blk,), lambda i: (i,))
    return pl.pallas_call(
        _combine_body,
        grid=(B // blk,),
        in_specs=[vec, vec, vec, vec, vec],
        out_specs=vec,
        out_shape=jax.ShapeDtypeStruct((B,), jnp.float32),
        name="combine",
    )(cont, *embs)


def kernel(x, x_cont, emb_x, table, w, bias):
    xt = x.T
    tab_t = table.T
    embs = []
    for f0, f1 in FIELD_SPLITS:
        if f1 % 4 == 0:
            row0 = 0
        else:
            row0 = f0 * FIELD_SIZE
        tab_k = lax.slice(tab_t, (0, row0), (1, f1 * FIELD_SIZE))
        embs.append(_emb_partial_sc(xt, tab_k, f0, f1, row0))
    cont = _matvec_tc(x_cont, w.reshape(1, 128), bias.reshape(1, 1))
    out = _combine_tc(cont, embs)
    return out.reshape(B, 1)


# speedup vs baseline: 2.4498x; 2.4498x over previous
import functools

import jax
import jax.numpy as jnp
from jax import lax
from jax.experimental import pallas as pl
from jax.experimental.pallas import tpu as pltpu
from jax.experimental.pallas import tpu_sc as plsc

B = 16384
NF = 26
FIELD_SIZE = 100000
NC = 2
NS = 16
NW = NC * NS
ROWS_W = B // NW
CH = 128
CPF = ROWS_W // CH
LANES = 16
DEPTH = 8

FIELD_SPLITS = ((8, 16), (0, 8), (20, 26), (16, 20))


def _emb_partial_sc(xt, tab, f0, f1, row0):
    nf = f1 - f0
    flat = nf * ROWS_W
    nch = flat // CH
    mesh = plsc.VectorSubcoreMesh(
        core_axis_name="c", subcore_axis_name="s", num_cores=NC, num_subcores=NS
    )

    @functools.partial(
        pl.kernel,
        out_type=jax.ShapeDtypeStruct((B,), jnp.float32),
        mesh=mesh,
        compiler_params=pltpu.CompilerParams(use_tc_tiling_on_sc=False),
        scratch_types=[
            pltpu.VMEM((nf, ROWS_W), jnp.int32),
            pltpu.VMEM((flat,), jnp.float32),
            pltpu.VMEM((ROWS_W,), jnp.float32),
            pltpu.SemaphoreType.DMA,
        ],
        name=f"emb_gather_f{f0}_{f1}",
    )
    def k(xt_hbm, tab_hbm, out_hbm, idx_v, rows_v, acc_v, sem):
        wid = lax.axis_index("s") * NC + lax.axis_index("c")
        base = wid * ROWS_W
        tab_flat = tab_hbm.at[0]
        pltpu.sync_copy(xt_hbm.at[pl.ds(f0, nf), pl.ds(base, ROWS_W)], idx_v)

        def add_offsets(f, carry):
            off = (f0 + f) * FIELD_SIZE - row0
            for c in range(ROWS_W // LANES):
                sl = pl.ds(c * LANES, LANES)
                idx_v[f, sl] = idx_v[f, sl] + off
            return carry

        lax.fori_loop(0, nf, add_offsets, 0)

        def src(j):
            return tab_flat.at[idx_v.at[j // CPF, pl.ds((j % CPF) * CH, CH)]]

        def fire(j):
            pltpu.async_copy(src(j), rows_v.at[pl.ds(j * CH, CH)], sem)

        def drain(j):
            pltpu.make_async_copy(src(j), rows_v.at[pl.ds(j * CH, CH)], sem).wait()

        for j in range(DEPTH):
            fire(j)

        def steady(j, carry):
            fire(j + DEPTH)
            drain(j)
            return carry

        lax.fori_loop(0, nch - DEPTH, steady, 0)

        def tail(j, carry):
            drain(j)
            return carry

        lax.fori_loop(nch - DEPTH, nch, tail, 0)

        def row_sum(g, carry):
            acc = rows_v[pl.ds(g * LANES, LANES)]
            for f in range(1, nf):
                acc = acc + rows_v[pl.ds(f * ROWS_W + g * LANES, LANES)]
            acc_v[pl.ds(g * LANES, LANES)] = acc
            return carry

        lax.fori_loop(0, ROWS_W // LANES, row_sum, 0)
        pltpu.sync_copy(acc_v, out_hbm.at[pl.ds(base, ROWS_W)])

    return k(xt, tab)


def _matvec_body(xc_ref, w_ref, b_ref, o_ref):
    o_ref[...] = jnp.sum(xc_ref[...] * w_ref[...], axis=1) + b_ref[0, 0]


def _matvec_tc(x_cont, w2, bias2):
    blk = 2048
    return pl.pallas_call(
        _matvec_body,
        grid=(B // blk,),
        in_specs=[
            pl.BlockSpec((blk, 128), lambda i: (i, 0)),
            pl.BlockSpec((1, 128), lambda i: (0, 0)),
            pl.BlockSpec((1, 1), lambda i: (0, 0)),
        ],
        out_specs=pl.BlockSpec((blk,), lambda i: (i,)),
        out_shape=jax.ShapeDtypeStruct((B,), jnp.float32),
        name="cont_matvec",
    )(x_cont, w2, bias2)


def _combine_body(c_ref, e0_ref, e1_ref, e2_ref, e3_ref, o_ref):
    o_ref[...] = c_ref[...] + e0_ref[...] + e1_ref[...] + e2_ref[...] + e3_ref[...]


def _combine_tc(cont, embs):
    blk = 4096
    vec = pl.BlockSpec((blk,), lambda i: (i,))
    return pl.pallas_call(
        _combine_body,
        grid=(B // blk,),
        in_specs=[vec, vec, vec, vec, vec],
        out_specs=vec,
        out_shape=jax.ShapeDtypeStruct((B,), jnp.float32),
        name="combine",
    )(cont, *embs)


def kernel(x, x_cont, emb_x, table, w, bias):
    xt = x.T
    tab_t = table.T
    embs = []
    for f0, f1 in FIELD_SPLITS:
        row0 = f0 * FIELD_SIZE
        tab_k = lax.slice(tab_t, (0, row0), (1, f1 * FIELD_SIZE))
        embs.append(_emb_partial_sc(xt, tab_k, f0, f1, row0))
    cont = _matvec_tc(x_cont, w.reshape(1, 128), bias.reshape(1, 1))
    out = _combine_tc(cont, embs)
    return out.reshape(B, 1)
